# Initial kernel scaffold; baseline (speedup 1.0000x reference)
import jax
import jax.numpy as jnp
from jax.experimental import pallas as pl


def kernel(x, weight, bias):
    raise NotImplementedError("write your pallas kernel here")


# speedup vs baseline: 1.9424x; 1.9424x over previous
import functools

import jax
import jax.numpy as jnp
from jax.experimental import pallas as pl
from jax.experimental.pallas import tpu as pltpu

N_TOKENS = 8192
DIM = 2048
N_EXPERTS = 64
TOP_K = 6

BT = 1024


def _gate_body(x_ref, w_ref, b_ref, wout_ref, iout_ref):
    x = x_ref[...]
    w = w_ref[...]
    s = jax.lax.dot_general(
        x, w, (((1,), (1,)), ((), ())), preferred_element_type=jnp.float32
    )
    m = jnp.max(s, axis=-1, keepdims=True)
    e = jnp.exp(s - m)
    p = e / jnp.sum(e, axis=-1, keepdims=True)
    biased = p + b_ref[...]
    iota = jax.lax.broadcasted_iota(jnp.int32, (x.shape[0], N_EXPERTS), 1)
    work = biased
    wcols, icols = [], []
    for _ in range(TOP_K):
        mx = jnp.max(work, axis=-1, keepdims=True)
        hit = work == mx
        widx = jnp.min(jnp.where(hit, iota, N_EXPERTS), axis=-1, keepdims=True)
        sel = iota == widx
        wval = jnp.sum(jnp.where(sel, p, 0.0), axis=-1, keepdims=True)
        work = jnp.where(sel, -jnp.inf, work)
        wcols.append(wval)
        icols.append(widx)
    wout_ref[...] = jnp.concatenate(wcols, axis=1)
    iout_ref[...] = jnp.concatenate(icols, axis=1)


@jax.jit
def kernel(x, weight, bias):
    n = x.shape[0]
    grid = (n // BT,)
    weights, indices = pl.pallas_call(
        _gate_body,
        grid=grid,
        in_specs=[
            pl.BlockSpec((BT, DIM), lambda i: (i, 0)),
            pl.BlockSpec((N_EXPERTS, DIM), lambda i: (0, 0)),
            pl.BlockSpec((1, N_EXPERTS), lambda i: (0, 0)),
        ],
        out_specs=[
            pl.BlockSpec((BT, TOP_K), lambda i: (i, 0)),
            pl.BlockSpec((BT, TOP_K), lambda i: (i, 0)),
        ],
        out_shape=[
            jax.ShapeDtypeStruct((n, TOP_K), jnp.float32),
            jax.ShapeDtypeStruct((n, TOP_K), jnp.int32),
        ],
    )(x, weight, bias.reshape(1, N_EXPERTS))
    return weights.astype(x.dtype), indices
